# baseline (speedup 1.0000x reference)
import jax
import jax.numpy as jnp
from jax import lax
from jax.experimental import pallas as pl
from jax.experimental.pallas import tpu as pltpu
from jax.experimental.pallas import tpu_sc as plsc

NC = 2
NS = 16
NW = NC * NS
B = 16384
D = 64
B_PER_W = B // NW


CHUNK = 256


def _sc_gather_body(uid_hbm, iid_hbm, ut_hbm, it_hbm, u_out, i_out,
                    uids_sm, iids_sm, urows, irows, sem):
  c = lax.axis_index("c")
  s = lax.axis_index("s")
  wid = s * NC + c
  base = wid * B_PER_W
  pltpu.sync_copy(uid_hbm.at[pl.ds(base, B_PER_W)], uids_sm)
  pltpu.sync_copy(iid_hbm.at[pl.ds(base, B_PER_W)], iids_sm)

  def gather_chunk(ids_ref, tab_hbm, rows_vmem, out_hbm, coff):
    def body(g, carry):
      goff = coff + g * 16
      vec = ids_ref[pl.ds(goff, 16)]
      for j in range(16):
        pltpu.async_copy(tab_hbm.at[vec[j]], rows_vmem.at[g * 16 + j], sem)
      return carry

    lax.fori_loop(0, CHUNK // 16, body, 0)
    pltpu.make_async_copy(tab_hbm.at[pl.ds(0, CHUNK)], rows_vmem, sem).wait()
    pltpu.sync_copy(rows_vmem, out_hbm.at[pl.ds(base + coff, CHUNK)])

  for cc in range(B_PER_W // CHUNK):
    gather_chunk(uids_sm, ut_hbm, urows, u_out, cc * CHUNK)
    gather_chunk(iids_sm, it_hbm, irows, i_out, cc * CHUNK)


_sc_gather = pl.kernel(
    _sc_gather_body,
    out_type=(
        jax.ShapeDtypeStruct((B, D), jnp.float32),
        jax.ShapeDtypeStruct((B, D), jnp.float32),
    ),
    mesh=plsc.VectorSubcoreMesh(core_axis_name="c", subcore_axis_name="s"),
    scratch_types=[
        pltpu.VMEM((B_PER_W,), jnp.int32),
        pltpu.VMEM((B_PER_W,), jnp.int32),
        pltpu.VMEM((CHUNK, D), jnp.float32),
        pltpu.VMEM((CHUNK, D), jnp.float32),
        pltpu.SemaphoreType.DMA,
    ],
)


BLK = 2048


def _mlp_body(u_ref, i_ref, w1u_ref, w1i_ref, b1_ref, w2_ref, b2_ref,
              w3_ref, b3_ref, o_ref):
  h = (jnp.dot(u_ref[...], w1u_ref[...], preferred_element_type=jnp.float32)
       + jnp.dot(i_ref[...], w1i_ref[...], preferred_element_type=jnp.float32)
       + b1_ref[...])
  h = jnp.maximum(h, 0.0)
  h = jnp.dot(h, w2_ref[...], preferred_element_type=jnp.float32) + b2_ref[...]
  h = jnp.maximum(h, 0.0)
  o_ref[...] = (jnp.dot(h, w3_ref[...], preferred_element_type=jnp.float32)
                + b3_ref[...])


_mlp = pl.pallas_call(
    _mlp_body,
    grid=(B // BLK,),
    in_specs=[
        pl.BlockSpec((BLK, D), lambda b: (b, 0)),
        pl.BlockSpec((BLK, D), lambda b: (b, 0)),
        pl.BlockSpec((D, 128), lambda b: (0, 0)),
        pl.BlockSpec((D, 128), lambda b: (0, 0)),
        pl.BlockSpec((1, 128), lambda b: (0, 0)),
        pl.BlockSpec((128, 64), lambda b: (0, 0)),
        pl.BlockSpec((1, 64), lambda b: (0, 0)),
        pl.BlockSpec((D, 1), lambda b: (0, 0)),
        pl.BlockSpec((1, 1), lambda b: (0, 0)),
    ],
    out_specs=pl.BlockSpec((BLK, 1), lambda b: (b, 0)),
    out_shape=jax.ShapeDtypeStruct((B, 1), jnp.float32),
)


@jax.jit
def kernel(user_ids, item_ids, user_table, item_table, W1, b1, W2, b2, W3, b3):
  u_e, i_e = _sc_gather(user_ids, item_ids, user_table, item_table)
  w1u = W1[:, :D].T
  w1i = W1[:, D:].T
  out = _mlp(u_e, i_e, w1u, w1i, b1[None, :], W2.T, b2[None, :],
             W3.T, b3[None, :])
  return out[:, 0]


# speedup vs baseline: 1.0141x; 1.0141x over previous
import jax
import jax.numpy as jnp
from jax import lax
from jax.experimental import pallas as pl
from jax.experimental.pallas import tpu as pltpu
from jax.experimental.pallas import tpu_sc as plsc

NC = 2
NS = 16
NW = NC * NS
B = 16384
D = 64
B_PER_W = B // NW


CHUNK = 256


def _sc_gather_body(ids_hbm, tab_hbm, out_hbm, ids_sm, rows, sem):
  c = lax.axis_index("c")
  s = lax.axis_index("s")
  wid = s * NC + c
  base = wid * B_PER_W
  pltpu.sync_copy(ids_hbm.at[pl.ds(base, B_PER_W)], ids_sm)

  def gather_chunk(coff):
    def body(g, carry):
      goff = coff + g * 16
      vec = ids_sm[pl.ds(goff, 16)]
      for j in range(16):
        pltpu.async_copy(tab_hbm.at[vec[j]], rows.at[g * 16 + j], sem)
      return carry

    lax.fori_loop(0, CHUNK // 16, body, 0)
    pltpu.make_async_copy(tab_hbm.at[pl.ds(0, CHUNK)], rows, sem).wait()
    pltpu.sync_copy(rows, out_hbm.at[pl.ds(base + coff, CHUNK)])

  for cc in range(B_PER_W // CHUNK):
    gather_chunk(cc * CHUNK)


_sc_gather = pl.kernel(
    _sc_gather_body,
    out_type=jax.ShapeDtypeStruct((B, D), jnp.float32),
    mesh=plsc.VectorSubcoreMesh(core_axis_name="c", subcore_axis_name="s"),
    scratch_types=[
        pltpu.VMEM((B_PER_W,), jnp.int32),
        pltpu.VMEM((CHUNK, D), jnp.float32),
        pltpu.SemaphoreType.DMA,
    ],
)


BLK = 2048


def _mlp_body(u_ref, i_ref, w1u_ref, w1i_ref, b1_ref, w2_ref, b2_ref,
              w3_ref, b3_ref, o_ref):
  h = (jnp.dot(u_ref[...], w1u_ref[...], preferred_element_type=jnp.float32)
       + jnp.dot(i_ref[...], w1i_ref[...], preferred_element_type=jnp.float32)
       + b1_ref[...])
  h = jnp.maximum(h, 0.0)
  h = jnp.dot(h, w2_ref[...], preferred_element_type=jnp.float32) + b2_ref[...]
  h = jnp.maximum(h, 0.0)
  o_ref[...] = (jnp.dot(h, w3_ref[...], preferred_element_type=jnp.float32)
                + b3_ref[...])


_mlp = pl.pallas_call(
    _mlp_body,
    grid=(B // BLK,),
    in_specs=[
        pl.BlockSpec((BLK, D), lambda b: (b, 0)),
        pl.BlockSpec((BLK, D), lambda b: (b, 0)),
        pl.BlockSpec((D, 128), lambda b: (0, 0)),
        pl.BlockSpec((D, 128), lambda b: (0, 0)),
        pl.BlockSpec((1, 128), lambda b: (0, 0)),
        pl.BlockSpec((128, 64), lambda b: (0, 0)),
        pl.BlockSpec((1, 64), lambda b: (0, 0)),
        pl.BlockSpec((D, 1), lambda b: (0, 0)),
        pl.BlockSpec((1, 1), lambda b: (0, 0)),
    ],
    out_specs=pl.BlockSpec((BLK, 1), lambda b: (b, 0)),
    out_shape=jax.ShapeDtypeStruct((B, 1), jnp.float32),
)


@jax.jit
def kernel(user_ids, item_ids, user_table, item_table, W1, b1, W2, b2, W3, b3):
  u_e = _sc_gather(user_ids, user_table)
  i_e = _sc_gather(item_ids, item_table)
  w1u = W1[:, :D].T
  w1i = W1[:, D:].T
  out = _mlp(u_e, i_e, w1u, w1i, b1[None, :], W2.T, b2[None, :],
             W3.T, b3[None, :])
  return out[:, 0]
